# baseline (speedup 1.0000x reference)
import functools

import jax
import jax.numpy as jnp
from jax import lax
from jax.experimental import pallas as pl
from jax.experimental.pallas import tpu as pltpu
from jax.experimental.pallas import tpu_sc as plsc


_N = 16384
_TN = 8 * 16384
_NW = 32
_PER_W = _TN // _NW
_VECS = _PER_W // 16


def _sc_max_body(tp_hbm, out_hbm, buf, part, sem):
    wid = lax.axis_index("s") * 2 + lax.axis_index("c")
    base = wid * _PER_W
    pltpu.async_copy(tp_hbm.at[pl.ds(base, _PER_W)], buf, sem).wait()

    def fold(j, acc):
        return jnp.maximum(acc, buf[pl.ds(j * 16, 16)])

    acc = lax.fori_loop(0, _VECS, fold, buf[pl.ds(0, 16)])
    part[...] = acc
    pltpu.sync_copy(part, out_hbm.at[pl.ds(wid * 16, 16)])


def _sc_max(trailing_pred):
    mesh = plsc.VectorSubcoreMesh(core_axis_name="c", subcore_axis_name="s")
    k = functools.partial(
        pl.kernel,
        mesh=mesh,
        out_type=jax.ShapeDtypeStruct((512,), jnp.float32),
        scratch_types=[
            pltpu.VMEM((_PER_W,), jnp.float32),
            pltpu.VMEM((16,), jnp.float32),
            pltpu.SemaphoreType.DMA,
        ],
    )(_sc_max_body)
    return k(trailing_pred)


def _tc_loss_kernel(part_ref, p_ref, g_ref, out_ref):
    thresh = jnp.max(part_ref[...])
    p = p_ref[...]
    g = g_ref[...]
    bce = g * jnp.log(p) + (1.0 - g) * jnp.log(1.0 - p)
    fltr = jnp.logical_and(p > thresh, g == 0.0)
    loss = jnp.where(fltr, bce * 20.0, bce)
    out_ref[...] = (jnp.sum(loss) * (1.0 / _N)).reshape(1, 1)


def kernel(prediction, ground, trailing_pred, trailing_ground):
    partials = _sc_max(trailing_pred).reshape(4, 128)
    p2 = prediction.reshape(128, 128)
    g2 = ground.reshape(128, 128)
    out = pl.pallas_call(
        _tc_loss_kernel,
        out_shape=jax.ShapeDtypeStruct((1, 1), jnp.float32),
    )(partials, p2, g2)
    return out[0, 0]
